# baseline (speedup 1.0000x reference)
import functools

import jax
import jax.numpy as jnp
from jax import lax
from jax.experimental import pallas as pl
from jax.experimental.pallas import tpu as pltpu
from jax.experimental.pallas import tpu_sc as plsc

_N = 16384
_C = 1000
_K = _N - int(0.75 * _N)
_BLK = 1024
_GRID = _N // _BLK

_NW = 32
_ROWS_PER_W = 128 // _NW


def _argmax_body(y_ref, out_ref):
    step = pl.program_id(0)
    y = y_ref[...]
    m = jnp.max(y, axis=1, keepdims=True)
    col = lax.broadcasted_iota(jnp.int32, (_BLK, _C), 1)
    idx = jnp.min(jnp.where(y == m, col, jnp.int32(_C)), axis=1)
    row = step * _BLK + lax.broadcasted_iota(jnp.int32, (_BLK,), 0)
    out_ref[...] = row * _C + idx


def _gather_sc_kernel(yhat_hbm, idx_hbm, out_hbm, idx_v, val_v, sem):
    wid = lax.axis_index("s") * 2 + lax.axis_index("c")
    r0 = wid * _ROWS_PER_W
    pltpu.sync_copy(idx_hbm.at[pl.ds(r0, _ROWS_PER_W), :], idx_v)
    copies = [
        pltpu.async_copy(yhat_hbm.at[idx_v.at[j]], val_v.at[j], sem)
        for j in range(_ROWS_PER_W)
    ]
    for c in copies:
        c.wait()
    pltpu.sync_copy(val_v, out_hbm.at[pl.ds(r0, _ROWS_PER_W), :])


_gather_sc = functools.partial(
    pl.kernel,
    out_type=jax.ShapeDtypeStruct((128, 128), jnp.float32),
    mesh=plsc.VectorSubcoreMesh(core_axis_name="c", subcore_axis_name="s"),
    scratch_types=[
        pltpu.VMEM((_ROWS_PER_W, 128), jnp.int32),
        pltpu.VMEM((_ROWS_PER_W, 128), jnp.float32),
        pltpu.SemaphoreType.DMA,
    ],
)(_gather_sc_kernel)


def _select_body(g_ref, out_ref):
    g = g_ref[...]
    b = lax.bitcast_convert_type(g, jnp.uint32)
    neg = (b >> jnp.uint32(31)) == jnp.uint32(1)
    key = jnp.where(neg, ~b, b | jnp.uint32(0x80000000))

    def body(i, prefix):
        t = prefix | (jnp.uint32(1) << (jnp.uint32(31) - i.astype(jnp.uint32)))
        cnt = jnp.sum((key < t).astype(jnp.int32))
        return jnp.where(cnt <= _K - 1, t, prefix)

    v = lax.fori_loop(0, 32, body, jnp.uint32(0))

    lt = key < v
    cnt_lt = jnp.sum(lt.astype(jnp.int32))
    sum_lt = jnp.sum(jnp.where(lt, g, jnp.float32(0.0)))
    vb = jnp.where((v >> jnp.uint32(31)) == jnp.uint32(1),
                   v ^ jnp.uint32(0x80000000), ~v)
    gv = lax.bitcast_convert_type(vb, jnp.float32)
    total = sum_lt + (jnp.float32(_K) - cnt_lt.astype(jnp.float32)) * gv
    out_ref[0, 0] = -total / jnp.float32(_K)


def kernel(y_hat, y):
    flat_idx = pl.pallas_call(
        _argmax_body,
        grid=(_GRID,),
        in_specs=[pl.BlockSpec((_BLK, _C), lambda i: (i, 0))],
        out_specs=pl.BlockSpec((_BLK,), lambda i: (i,)),
        out_shape=jax.ShapeDtypeStruct((_N,), jnp.int32),
        compiler_params=pltpu.CompilerParams(
            dimension_semantics=("arbitrary",),
        ),
    )(y)
    g2d = _gather_sc(y_hat.reshape(-1), flat_idx.reshape(128, 128))
    out = pl.pallas_call(
        _select_body,
        in_specs=[pl.BlockSpec((128, 128), lambda: (0, 0))],
        out_specs=pl.BlockSpec(memory_space=pltpu.SMEM),
        out_shape=jax.ShapeDtypeStruct((1, 1), jnp.float32),
    )(g2d)
    return out[0, 0]


# speedup vs baseline: 1.3349x; 1.3349x over previous
import functools

import jax
import jax.numpy as jnp
from jax import lax
from jax.experimental import pallas as pl
from jax.experimental.pallas import tpu as pltpu

_N = 16384
_C = 1000
_K = _N - int(0.75 * _N)
_BLK = 2048
_GRID = _N // _BLK


def _select_loss(g):
    b = lax.bitcast_convert_type(g, jnp.uint32)
    neg = (b >> jnp.uint32(31)) == jnp.uint32(1)
    key = jnp.where(neg, ~b, b | jnp.uint32(0x80000000))

    def body(i, prefix):
        t = prefix | (jnp.uint32(1) << (jnp.uint32(31) - i.astype(jnp.uint32)))
        cnt = jnp.sum((key < t).astype(jnp.int32))
        return jnp.where(cnt <= _K - 1, t, prefix)

    v = lax.fori_loop(0, 32, body, jnp.uint32(0))

    lt = key < v
    cnt_lt = jnp.sum(lt.astype(jnp.int32))
    sum_lt = jnp.sum(jnp.where(lt, g, jnp.float32(0.0)))
    vb = jnp.where((v >> jnp.uint32(31)) == jnp.uint32(1),
                   v ^ jnp.uint32(0x80000000), ~v)
    gv = lax.bitcast_convert_type(vb, jnp.float32)
    total = sum_lt + (jnp.float32(_K) - cnt_lt.astype(jnp.float32)) * gv
    return -total / jnp.float32(_K)


def _fused_body(y_hat_ref, y_ref, out_ref, g_ref):
    step = pl.program_id(0)
    y = y_ref[...]
    yh = y_hat_ref[...]
    m = jnp.max(y, axis=1, keepdims=True)
    col = lax.broadcasted_iota(jnp.int32, (_BLK, _C), 1)
    idx = jnp.min(jnp.where(y == m, col, jnp.int32(_C)), axis=1, keepdims=True)
    g2 = jnp.where(col == idx, yh, jnp.float32(0.0))
    g = jnp.sum(g2, axis=1)
    g_ref[pl.ds(step * _BLK, _BLK)] = g

    @pl.when(step == _GRID - 1)
    def _():
        out_ref[0, 0] = _select_loss(g_ref[...])


def kernel(y_hat, y):
    out = pl.pallas_call(
        _fused_body,
        grid=(_GRID,),
        in_specs=[
            pl.BlockSpec((_BLK, _C), lambda i: (i, 0)),
            pl.BlockSpec((_BLK, _C), lambda i: (i, 0)),
        ],
        out_specs=pl.BlockSpec(memory_space=pltpu.SMEM),
        out_shape=jax.ShapeDtypeStruct((1, 1), jnp.float32),
        scratch_shapes=[pltpu.VMEM((_N,), jnp.float32)],
        compiler_params=pltpu.CompilerParams(
            dimension_semantics=("arbitrary",),
        ),
    )(y_hat, y)
    return out[0, 0]
